# baseline (speedup 1.0000x reference)
import functools

import jax
import jax.numpy as jnp
from jax import lax
from jax.experimental import pallas as pl
from jax.experimental.pallas import tpu as pltpu
from jax.experimental.pallas import tpu_sc as plsc

B = 16384
D = 64
MARGIN = 1.0

NC = 2
NS = 16
NW = NC * NS
PER_W = B // NW
CHUNK = 128
NCHUNK = PER_W // CHUNK

_mesh = plsc.VectorSubcoreMesh(core_axis_name="c", subcore_axis_name="s")


@functools.partial(
    pl.kernel,
    out_type=[jax.ShapeDtypeStruct((B, D), jnp.float32) for _ in range(6)],
    mesh=_mesh,
    scratch_types=[
        pltpu.VMEM((CHUNK,), jnp.int32),
        pltpu.VMEM((CHUNK, D), jnp.float32),
        pltpu.SemaphoreType.DMA,
    ],
    compiler_params=pltpu.CompilerParams(use_tc_tiling_on_sc=False),
)
def _sc_gather(ent_hbm, rel_hbm, ph_hbm, pr_hbm, pt_hbm, nh_hbm, nr_hbm,
               nt_hbm, o_ph, o_pr, o_pt, o_nh, o_nr, o_nt, idx_v, rows_v,
               sem):
    wid = lax.axis_index("s") * NC + lax.axis_index("c")
    triples = [
        (ent_hbm, ph_hbm, o_ph),
        (rel_hbm, pr_hbm, o_pr),
        (ent_hbm, pt_hbm, o_pt),
        (ent_hbm, nh_hbm, o_nh),
        (rel_hbm, nr_hbm, o_nr),
        (ent_hbm, nt_hbm, o_nt),
    ]
    for table, idx_hbm, out_hbm in triples:
        @pl.loop(0, NCHUNK)
        def _(ci, table=table, idx_hbm=idx_hbm, out_hbm=out_hbm):
            base = wid * PER_W + ci * CHUNK
            pltpu.sync_copy(idx_hbm.at[pl.ds(base, CHUNK)], idx_v)
            pltpu.async_copy(table.at[idx_v], rows_v, sem).wait()
            pltpu.sync_copy(rows_v, out_hbm.at[pl.ds(base, CHUNK)])


BLK = 1024
GRID = B // BLK


def _tc_body(ph, pr, pt, nh, nr, nt, out):
    i = pl.program_id(0)
    pos = ph[...] + pr[...] - pt[...]
    neg = nh[...] + nr[...] - nt[...]
    pd = jnp.sqrt(jnp.sum(pos * pos, axis=1))
    nd = jnp.sqrt(jnp.sum(neg * neg, axis=1))
    part = jnp.sum(jnp.maximum(pd - nd + MARGIN, 0.0))

    @pl.when(i == 0)
    def _():
        out[0] = 0.0

    out[0] += part

    @pl.when(i == GRID - 1)
    def _():
        out[0] = out[0] / B


_tc_loss = pl.pallas_call(
    _tc_body,
    grid=(GRID,),
    in_specs=[pl.BlockSpec((BLK, D), lambda i: (i, 0))] * 6,
    out_specs=pl.BlockSpec(memory_space=pltpu.SMEM),
    out_shape=jax.ShapeDtypeStruct((1,), jnp.float32),
)


def kernel(pos_h, pos_r, pos_t, neg_h, neg_r, neg_t, entity_emb,
           relation_emb):
    ph = pos_h.astype(jnp.int32)
    pr = pos_r.astype(jnp.int32)
    pt = pos_t.astype(jnp.int32)
    nh = neg_h.astype(jnp.int32)
    nr = neg_r.astype(jnp.int32)
    nt = neg_t.astype(jnp.int32)
    g_ph, g_pr, g_pt, g_nh, g_nr, g_nt = _sc_gather(
        entity_emb, relation_emb, ph, pr, pt, nh, nr, nt)
    loss = _tc_loss(g_ph, g_pr, g_pt, g_nh, g_nr, g_nt)
    return loss[0]
